# baseline (speedup 1.0000x reference)
import functools

import jax
import jax.numpy as jnp
from jax import lax
from jax.experimental import pallas as pl
from jax.experimental.pallas import tpu as pltpu
from jax.experimental.pallas import tpu_sc as plsc

_NUM_FIELDS = 26
_VOCAB = 100000
_EMBED_DIM = 128
_BATCH = 4096
_TOTAL = _BATCH * _NUM_FIELDS
_NC = 2
_NS = 16
_NW = _NC * _NS
_PER_W = _TOTAL // _NW
_CHUNK = 128
_NCHUNK = _PER_W // _CHUNK
_LANES = 16


def _body(x_hbm, tab_hbm, out_hbm, idx_v, rows_v, sem):
    wid = lax.axis_index("s") * _NC + lax.axis_index("c")
    base = wid * _PER_W

    def chunk_body(c, carry):
        gb = base + c * _CHUNK
        pltpu.sync_copy(x_hbm.at[pl.ds(gb, _CHUNK)], idx_v)

        def lane_body(i, carry2):
            q = c * _CHUNK + i * _LANES + lax.iota(jnp.int32, _LANES)
            off = (q % _NUM_FIELDS) * _VOCAB
            sl = pl.ds(i * _LANES, _LANES)
            idx_v[sl] = idx_v[sl] + off
            return carry2

        lax.fori_loop(0, _CHUNK // _LANES, lane_body, 0)
        pltpu.async_copy(tab_hbm.at[idx_v], rows_v, sem).wait()
        pltpu.sync_copy(rows_v, out_hbm.at[pl.ds(gb, _CHUNK)])
        return carry

    lax.fori_loop(0, _NCHUNK, chunk_body, 0)


def kernel(x, tables):
    xflat = x.reshape(_TOTAL)
    tab2d = tables.reshape(_NUM_FIELDS * _VOCAB, _EMBED_DIM)
    mesh = plsc.VectorSubcoreMesh(core_axis_name="c", subcore_axis_name="s")
    k = functools.partial(
        pl.kernel,
        mesh=mesh,
        out_type=jax.ShapeDtypeStruct((_TOTAL, _EMBED_DIM), jnp.float32),
        scratch_types=[
            pltpu.VMEM((_CHUNK,), jnp.int32),
            pltpu.VMEM((_CHUNK, _EMBED_DIM), jnp.float32),
            pltpu.SemaphoreType.DMA,
        ],
    )(_body)
    out = k(xflat, tab2d)
    return out.reshape(_BATCH, _NUM_FIELDS, _EMBED_DIM)


# speedup vs baseline: 1.1482x; 1.1482x over previous
import functools

import jax
import jax.numpy as jnp
from jax import lax
from jax.experimental import pallas as pl
from jax.experimental.pallas import tpu as pltpu
from jax.experimental.pallas import tpu_sc as plsc

_NUM_FIELDS = 26
_VOCAB = 100000
_EMBED_DIM = 128
_BATCH = 4096
_TOTAL = _BATCH * _NUM_FIELDS
_NC = 2
_NS = 16
_NW = _NC * _NS
_PER_W = _TOTAL // _NW
_CHUNK = 128
_NCHUNK = _PER_W // _CHUNK
_LANES = 16


def _body(x_hbm, tab_hbm, out_hbm, idx_v, rows_a, rows_b, gsa, gsb, wsa, wsb):
    wid = lax.axis_index("s") * _NC + lax.axis_index("c")
    base = wid * _PER_W

    pltpu.sync_copy(x_hbm.at[wid], idx_v)

    def off_body(c, carry):
        for i in range(_CHUNK // _LANES):
            q = c * _CHUNK + i * _LANES + lax.iota(jnp.int32, _LANES)
            sl = pl.ds(i * _LANES, _LANES)
            idx_v[c, sl] = idx_v[c, sl] + (q % _NUM_FIELDS) * _VOCAB
        return carry

    lax.fori_loop(0, _NCHUNK, off_body, 0)

    def wait_wb(buf, sem, c):
        pltpu.make_async_copy(buf, out_hbm.at[pl.ds(base + c * _CHUNK, _CHUNK)],
                              sem).wait()

    def pair_body(g, carry):
        c0 = g * 2
        c1 = c0 + 1

        @pl.when(g >= 1)
        def _():
            wait_wb(rows_a, wsa, c0)

        ga = pltpu.async_copy(tab_hbm.at[idx_v.at[c0]], rows_a, gsa)

        @pl.when(g >= 1)
        def _():
            wait_wb(rows_b, wsb, c1)

        gb = pltpu.async_copy(tab_hbm.at[idx_v.at[c1]], rows_b, gsb)

        ga.wait()
        pltpu.async_copy(rows_a, out_hbm.at[pl.ds(base + c0 * _CHUNK, _CHUNK)],
                         wsa)
        gb.wait()
        pltpu.async_copy(rows_b, out_hbm.at[pl.ds(base + c1 * _CHUNK, _CHUNK)],
                         wsb)
        return carry

    lax.fori_loop(0, _NCHUNK // 2, pair_body, 0)
    wait_wb(rows_a, wsa, 0)
    wait_wb(rows_b, wsb, 1)


def kernel(x, tables):
    x3d = x.reshape(_NW, _NCHUNK, _CHUNK)
    tab2d = tables.reshape(_NUM_FIELDS * _VOCAB, _EMBED_DIM)
    mesh = plsc.VectorSubcoreMesh(core_axis_name="c", subcore_axis_name="s")
    k = functools.partial(
        pl.kernel,
        mesh=mesh,
        out_type=jax.ShapeDtypeStruct((_TOTAL, _EMBED_DIM), jnp.float32),
        scratch_types=[
            pltpu.VMEM((_NCHUNK, _CHUNK), jnp.int32),
            pltpu.VMEM((_CHUNK, _EMBED_DIM), jnp.float32),
            pltpu.VMEM((_CHUNK, _EMBED_DIM), jnp.float32),
            pltpu.SemaphoreType.DMA,
            pltpu.SemaphoreType.DMA,
            pltpu.SemaphoreType.DMA,
            pltpu.SemaphoreType.DMA,
        ],
    )(_body)
    out = k(x3d, tab2d)
    return out.reshape(_BATCH, _NUM_FIELDS, _EMBED_DIM)
